# baseline (speedup 1.0000x reference)
import math

import jax
import jax.numpy as jnp
from jax.experimental import pallas as pl
from jax.experimental.pallas import tpu as pltpu

_V = 100000
_PADDING_IDX = 1
_SMOOTHING = 0.1
_CONF = 1.0 - _SMOOTHING
_EPS = _SMOOTHING / _V
_C1 = (_V - 1) * _EPS * math.log(_EPS) + _CONF * math.log(_CONF)

_N = 512
_VB = 4096
_NB = -(-_V // _VB)
_RG = 8


def _stats_body(x_ref, logz_ref, sx_ref, m_sc, s_sc, sx_sc):
    j = pl.program_id(0)

    @pl.when(j == 0)
    def _init():
        m_sc[...] = jnp.full((_N, 1), -jnp.inf, jnp.float32)
        s_sc[...] = jnp.zeros((_N, 1), jnp.float32)
        sx_sc[...] = jnp.zeros((_N, 1), jnp.float32)

    x = x_ref[...]

    @pl.when(j < _NB - 1)
    def _full():
        m0 = m_sc[...]
        mn = jnp.maximum(m0, jnp.max(x, axis=1, keepdims=True))
        s_sc[...] = (s_sc[...] * jnp.exp(m0 - mn)
                     + jnp.sum(jnp.exp(x - mn), axis=1, keepdims=True))
        m_sc[...] = mn
        sx_sc[...] += jnp.sum(x, axis=1, keepdims=True)

    @pl.when(j == _NB - 1)
    def _tail():
        lane = jax.lax.broadcasted_iota(jnp.int32, (_N, _VB), 1)
        valid = lane < (_V - j * _VB)
        xm = jnp.where(valid, x, -jnp.inf)
        m0 = m_sc[...]
        mn = jnp.maximum(m0, jnp.max(xm, axis=1, keepdims=True))
        s = (s_sc[...] * jnp.exp(m0 - mn)
             + jnp.sum(jnp.exp(xm - mn), axis=1, keepdims=True))
        logz_ref[...] = mn + jnp.log(s)
        sx_ref[...] = sx_sc[...] + jnp.sum(jnp.where(valid, x, 0.0),
                                           axis=1, keepdims=True)


def _xt_body(t_s, *refs):
    *b_refs, out_ref = refs
    i = pl.program_id(0)
    lane = jax.lax.broadcasted_iota(jnp.int32, (_RG, 128), 1)
    row = jax.lax.broadcasted_iota(jnp.int32, (_RG, 128), 0)
    acc = jnp.zeros((_RG, 128), jnp.float32)
    for k in range(_RG):
        lk = jnp.bitwise_and(t_s[i * _RG + k], 127)
        acc = acc + jnp.where((row == k) & (lane == lk), b_refs[k][...], 0.0)
    out_ref[...] = jnp.sum(acc, axis=1, keepdims=True)


def _finalize_body(logz_ref, sx_ref, xt_ref, t_ref, dl_ref, out_ref):
    logz = logz_ref[...]
    denom = jnp.sum(dl_ref[...], axis=0, keepdims=True)
    row_kl = (jnp.float32(_C1)
              - jnp.float32(_EPS) * (sx_ref[...]
                                     - jnp.float32(_V) * logz)
              - jnp.float32(_CONF - _EPS) * (xt_ref[...] - logz))
    row_kl = jnp.where(t_ref[...] == _PADDING_IDX, 0.0, row_kl)
    out_ref[...] = jnp.sum(row_kl, axis=0, keepdims=True) / denom


def kernel(pred, targets, decode_lengths):
    x = pred.reshape(_N, _V)
    t1 = targets.reshape(_N).astype(jnp.int32)
    t2 = t1.reshape(_N, 1)
    dl = decode_lengths.reshape(-1, 1).astype(jnp.float32)

    logz, sx = pl.pallas_call(
        _stats_body,
        grid=(_NB,),
        in_specs=[
            pl.BlockSpec((_N, _VB), lambda j: (0, j)),
        ],
        out_specs=[
            pl.BlockSpec((_N, 1), lambda j: (0, 0)),
            pl.BlockSpec((_N, 1), lambda j: (0, 0)),
        ],
        out_shape=[
            jax.ShapeDtypeStruct((_N, 1), jnp.float32),
            jax.ShapeDtypeStruct((_N, 1), jnp.float32),
        ],
        scratch_shapes=[
            pltpu.VMEM((_N, 1), jnp.float32),
            pltpu.VMEM((_N, 1), jnp.float32),
            pltpu.VMEM((_N, 1), jnp.float32),
        ],
        compiler_params=pltpu.CompilerParams(
            dimension_semantics=("arbitrary",),
        ),
    )(x)

    def _mk_spec(k):
        return pl.BlockSpec(
            (_RG, 128), lambda i, ts, k=k: (i, ts[i * _RG + k] // 128))

    xt = pl.pallas_call(
        _xt_body,
        grid_spec=pltpu.PrefetchScalarGridSpec(
            num_scalar_prefetch=1,
            grid=(_N // _RG,),
            in_specs=[_mk_spec(k) for k in range(_RG)],
            out_specs=pl.BlockSpec((_RG, 1), lambda i, ts: (i, 0)),
        ),
        out_shape=jax.ShapeDtypeStruct((_N, 1), jnp.float32),
    )(t1, *([x] * _RG))

    out = pl.pallas_call(
        _finalize_body,
        out_shape=jax.ShapeDtypeStruct((1, 1), jnp.float32),
    )(logz, sx, xt, t2, dl)
    return out.reshape(())


# speedup vs baseline: 1.1480x; 1.1480x over previous
import math

import jax
import jax.numpy as jnp
from jax.experimental import pallas as pl
from jax.experimental.pallas import tpu as pltpu

_V = 100000
_PADDING_IDX = 1
_SMOOTHING = 0.1
_CONF = 1.0 - _SMOOTHING
_EPS = _SMOOTHING / _V
_C1 = (_V - 1) * _EPS * math.log(_EPS) + _CONF * math.log(_CONF)

_N = 512
_VB = 4096
_NB = -(-_V // _VB)
_RG = 32


def _stats_body(x_ref, logz_ref, sx_ref, m_sc, s_sc, sx_sc):
    j = pl.program_id(0)

    @pl.when(j == 0)
    def _init():
        m_sc[...] = jnp.full((_N, 1), -jnp.inf, jnp.float32)
        s_sc[...] = jnp.zeros((_N, 1), jnp.float32)
        sx_sc[...] = jnp.zeros((_N, 1), jnp.float32)

    x = x_ref[...]

    @pl.when(j < _NB - 1)
    def _full():
        m0 = m_sc[...]
        mn = jnp.maximum(m0, jnp.max(x, axis=1, keepdims=True))
        s_sc[...] = (s_sc[...] * jnp.exp(m0 - mn)
                     + jnp.sum(jnp.exp(x - mn), axis=1, keepdims=True))
        m_sc[...] = mn
        sx_sc[...] += jnp.sum(x, axis=1, keepdims=True)

    @pl.when(j == _NB - 1)
    def _tail():
        lane = jax.lax.broadcasted_iota(jnp.int32, (_N, _VB), 1)
        valid = lane < (_V - j * _VB)
        xm = jnp.where(valid, x, -jnp.inf)
        m0 = m_sc[...]
        mn = jnp.maximum(m0, jnp.max(xm, axis=1, keepdims=True))
        s = (s_sc[...] * jnp.exp(m0 - mn)
             + jnp.sum(jnp.exp(xm - mn), axis=1, keepdims=True))
        logz_ref[...] = mn + jnp.log(s)
        sx_ref[...] = sx_sc[...] + jnp.sum(jnp.where(valid, x, 0.0),
                                           axis=1, keepdims=True)


def _xt_body(t_s, *refs):
    *b_refs, out_ref = refs
    i = pl.program_id(0)
    lane = jax.lax.broadcasted_iota(jnp.int32, (_RG, 128), 1)
    row = jax.lax.broadcasted_iota(jnp.int32, (_RG, 128), 0)
    acc = jnp.zeros((_RG, 128), jnp.float32)
    for k in range(_RG):
        lk = jnp.bitwise_and(t_s[i * _RG + k], 127)
        acc = acc + jnp.where((row == k) & (lane == lk), b_refs[k][...], 0.0)
    out_ref[...] = jnp.sum(acc, axis=1, keepdims=True)


def _finalize_body(logz_ref, sx_ref, xt_ref, t_ref, dl_ref, out_ref):
    logz = logz_ref[...]
    denom = jnp.sum(dl_ref[...], axis=0, keepdims=True)
    row_kl = (jnp.float32(_C1)
              - jnp.float32(_EPS) * (sx_ref[...]
                                     - jnp.float32(_V) * logz)
              - jnp.float32(_CONF - _EPS) * (xt_ref[...] - logz))
    row_kl = jnp.where(t_ref[...] == _PADDING_IDX, 0.0, row_kl)
    out_ref[...] = jnp.sum(row_kl, axis=0, keepdims=True) / denom


def kernel(pred, targets, decode_lengths):
    x = pred.reshape(_N, _V)
    t1 = targets.reshape(_N).astype(jnp.int32)
    t2 = t1.reshape(_N, 1)
    dl = decode_lengths.reshape(-1, 1).astype(jnp.float32)

    logz, sx = pl.pallas_call(
        _stats_body,
        grid=(_NB,),
        in_specs=[
            pl.BlockSpec((_N, _VB), lambda j: (0, j)),
        ],
        out_specs=[
            pl.BlockSpec((_N, 1), lambda j: (0, 0)),
            pl.BlockSpec((_N, 1), lambda j: (0, 0)),
        ],
        out_shape=[
            jax.ShapeDtypeStruct((_N, 1), jnp.float32),
            jax.ShapeDtypeStruct((_N, 1), jnp.float32),
        ],
        scratch_shapes=[
            pltpu.VMEM((_N, 1), jnp.float32),
            pltpu.VMEM((_N, 1), jnp.float32),
            pltpu.VMEM((_N, 1), jnp.float32),
        ],
        compiler_params=pltpu.CompilerParams(
            dimension_semantics=("arbitrary",),
        ),
    )(x)

    def _mk_spec(k):
        return pl.BlockSpec(
            (_RG, 128), lambda i, ts, k=k: (i, ts[i * _RG + k] // 128))

    xt = pl.pallas_call(
        _xt_body,
        grid_spec=pltpu.PrefetchScalarGridSpec(
            num_scalar_prefetch=1,
            grid=(_N // _RG,),
            in_specs=[_mk_spec(k) for k in range(_RG)],
            out_specs=pl.BlockSpec((_RG, 1), lambda i, ts: (i, 0)),
        ),
        out_shape=jax.ShapeDtypeStruct((_N, 1), jnp.float32),
    )(t1, *([x] * _RG))

    out = pl.pallas_call(
        _finalize_body,
        out_shape=jax.ShapeDtypeStruct((1, 1), jnp.float32),
    )(logz, sx, xt, t2, dl)
    return out.reshape(())
